# baseline (speedup 1.0000x reference)
import functools

import jax
import jax.numpy as jnp
from jax import lax
from jax.experimental import pallas as pl
from jax.experimental.pallas import tpu as pltpu
from jax.experimental.pallas import tpu_sc as plsc

EMBED_DIM = 64
PAD_DIM = 128
TBLOCK = 512


@functools.lru_cache(maxsize=None)
def _make_transpose(n_items):
    nblocks = (n_items + TBLOCK - 1) // TBLOCK
    n_rows_out = nblocks * TBLOCK

    def body(t_ref, out_ref):
        out_ref[:, 0:EMBED_DIM] = t_ref[...].T

    return pl.pallas_call(
        body,
        grid=(nblocks,),
        in_specs=[pl.BlockSpec((EMBED_DIM, TBLOCK), lambda i: (0, i))],
        out_specs=pl.BlockSpec((TBLOCK, PAD_DIM), lambda i: (i, 0)),
        out_shape=jax.ShapeDtypeStruct((n_rows_out, PAD_DIM), jnp.float32),
    )


@functools.lru_cache(maxsize=None)
def _make_gather(n_rows, table_rows, n_workers, chunk):
    b_per_w = n_rows // n_workers
    nchunks = b_per_w // chunk
    nbuf = 2
    mesh = plsc.VectorSubcoreMesh(core_axis_name="c", subcore_axis_name="s")

    @functools.partial(
        pl.kernel,
        mesh=mesh,
        out_type=jax.ShapeDtypeStruct((n_rows, EMBED_DIM), jnp.float32),
        compiler_params=pltpu.CompilerParams(use_tc_tiling_on_sc=False),
        scratch_types=[
            pltpu.VMEM((b_per_w,), jnp.int32),
            pltpu.VMEM((chunk, PAD_DIM), jnp.float32),
            pltpu.VMEM((chunk, PAD_DIM), jnp.float32),
            pltpu.SemaphoreType.DMA,
            pltpu.SemaphoreType.DMA,
            pltpu.SemaphoreType.DMA,
            pltpu.SemaphoreType.DMA,
        ],
    )
    def k(table_hbm, idx_hbm, out_hbm, idx_v, buf0, buf1, gsem0, gsem1, osem0, osem1):
        wid = lax.axis_index("s") * 2 + lax.axis_index("c")
        base = wid * b_per_w
        pltpu.sync_copy(idx_hbm.at[pl.ds(base, b_per_w)], idx_v)
        bufs = (buf0, buf1)
        gsems = (gsem0, gsem1)
        osems = (osem0, osem1)
        gcopies = [None] * nchunks
        ocopies = [None] * nchunks

        def start_out(ci):
            s = ci % nbuf
            return pltpu.async_copy(
                bufs[s].at[:, pl.ds(0, EMBED_DIM)],
                out_hbm.at[pl.ds(base + ci * chunk, chunk)],
                osems[s],
            )

        for ci in range(nchunks):
            s = ci % nbuf
            if ci >= nbuf:
                ocopies[ci - nbuf].wait()
            gcopies[ci] = pltpu.async_copy(
                table_hbm.at[idx_v.at[pl.ds(ci * chunk, chunk)]],
                bufs[s],
                gsems[s],
            )
            if ci >= 1:
                gcopies[ci - 1].wait()
                ocopies[ci - 1] = start_out(ci - 1)
        gcopies[nchunks - 1].wait()
        ocopies[nchunks - 1] = start_out(nchunks - 1)
        ocopies[nchunks - 2].wait()
        ocopies[nchunks - 1].wait()

    return k


def kernel(item_ids, item_emb):
    b, h = item_ids.shape
    n_rows = b * h
    n_items = item_emb.shape[0]
    ids = item_ids.reshape(n_rows).astype(jnp.int32)
    t128 = _make_transpose(n_items)(item_emb.T)
    out = _make_gather(n_rows, t128.shape[0], 32, 400)(t128, ids)
    return out.reshape(b, h, EMBED_DIM)


# speedup vs baseline: 2.9672x; 2.9672x over previous
import functools

import jax
import jax.numpy as jnp
from jax import lax
from jax.experimental import pallas as pl
from jax.experimental.pallas import tpu as pltpu
from jax.experimental.pallas import tpu_sc as plsc

EMBED_DIM = 64
PAD_DIM = 128
TBLOCK = 8192


@functools.lru_cache(maxsize=None)
def _make_transpose(n_items):
    nblocks = (n_items + TBLOCK - 1) // TBLOCK
    n_rows_out = nblocks * TBLOCK

    def body(t_ref, out_ref):
        out_ref[:, 0:EMBED_DIM] = t_ref[...].T

    return pl.pallas_call(
        body,
        grid=(nblocks,),
        in_specs=[pl.BlockSpec((EMBED_DIM, TBLOCK), lambda i: (0, i))],
        out_specs=pl.BlockSpec((TBLOCK, PAD_DIM), lambda i: (i, 0)),
        out_shape=jax.ShapeDtypeStruct((n_rows_out, PAD_DIM), jnp.float32),
    )


@functools.lru_cache(maxsize=None)
def _make_gather(n_rows, table_rows, n_workers, chunk):
    b_per_w = n_rows // n_workers
    nchunks = b_per_w // chunk
    nbuf = 2
    mesh = plsc.VectorSubcoreMesh(core_axis_name="c", subcore_axis_name="s")

    @functools.partial(
        pl.kernel,
        mesh=mesh,
        out_type=jax.ShapeDtypeStruct((n_rows, EMBED_DIM), jnp.float32),
        compiler_params=pltpu.CompilerParams(use_tc_tiling_on_sc=False),
        scratch_types=[
            pltpu.VMEM((b_per_w,), jnp.int32),
            pltpu.VMEM((chunk, PAD_DIM), jnp.float32),
            pltpu.VMEM((chunk, PAD_DIM), jnp.float32),
            pltpu.SemaphoreType.DMA,
            pltpu.SemaphoreType.DMA,
            pltpu.SemaphoreType.DMA,
            pltpu.SemaphoreType.DMA,
        ],
    )
    def k(table_hbm, idx_hbm, out_hbm, idx_v, buf0, buf1, gsem0, gsem1, osem0, osem1):
        wid = lax.axis_index("s") * 2 + lax.axis_index("c")
        base = wid * b_per_w
        pltpu.sync_copy(idx_hbm.at[pl.ds(base, b_per_w)], idx_v)
        bufs = (buf0, buf1)
        gsems = (gsem0, gsem1)
        osems = (osem0, osem1)
        gcopies = [None] * nchunks
        ocopies = [None] * nchunks

        def start_out(ci):
            s = ci % nbuf
            return pltpu.async_copy(
                bufs[s].at[:, pl.ds(0, EMBED_DIM)],
                out_hbm.at[pl.ds(base + ci * chunk, chunk)],
                osems[s],
            )

        for ci in range(nchunks):
            s = ci % nbuf
            if ci >= nbuf:
                ocopies[ci - nbuf].wait()
            gcopies[ci] = pltpu.async_copy(
                table_hbm.at[idx_v.at[pl.ds(ci * chunk, chunk)]],
                bufs[s],
                gsems[s],
            )
            if ci >= 1:
                gcopies[ci - 1].wait()
                ocopies[ci - 1] = start_out(ci - 1)
        gcopies[nchunks - 1].wait()
        ocopies[nchunks - 1] = start_out(nchunks - 1)
        ocopies[nchunks - 2].wait()
        ocopies[nchunks - 1].wait()

    return k


def kernel(item_ids, item_emb):
    b, h = item_ids.shape
    n_rows = b * h
    n_items = item_emb.shape[0]
    ids = item_ids.reshape(n_rows).astype(jnp.int32)
    t128 = _make_transpose(n_items)(item_emb.T)
    out = _make_gather(n_rows, t128.shape[0], 32, 400)(t128, ids)
    return out.reshape(b, h, EMBED_DIM)
